# baseline (speedup 1.0000x reference)
import functools

import jax
import jax.numpy as jnp
from jax import lax
from jax.experimental import pallas as pl
from jax.experimental.pallas import tpu as pltpu
from jax.experimental.pallas import tpu_sc as plsc

B = 16384
D = 128
NC = 2
NS = 16
NW = NC * NS
BPW = B // NW
CH = 128
NCHUNK = BPW // CH
NG = CH // 16


def _sc_energies_body(xs_hbm, xr_hbm, xo_hbm, subj_hbm, rel_hbm, obj_hbm,
                      out_hbm, xs_v, xr_v, xo_v, s_v, r_v, o_v, e_v, sem):
    wid = lax.axis_index("s") * NC + lax.axis_index("c")
    base = wid * BPW
    pltpu.sync_copy(xs_hbm.at[pl.ds(base, BPW)], xs_v)
    pltpu.sync_copy(xr_hbm.at[pl.ds(base, BPW)], xr_v)
    pltpu.sync_copy(xo_hbm.at[pl.ds(base, BPW)], xo_v)
    iota = lax.iota(jnp.int32, 16)

    def chunk_body(c, carry):
        off = c * CH
        cp_s = pltpu.async_copy(subj_hbm.at[xs_v.at[pl.ds(off, CH)]], s_v, sem)
        cp_r = pltpu.async_copy(rel_hbm.at[xr_v.at[pl.ds(off, CH)]], r_v, sem)
        cp_o = pltpu.async_copy(obj_hbm.at[xo_v.at[pl.ds(off, CH)]], o_v, sem)
        cp_s.wait()
        cp_r.wait()
        cp_o.wait()

        def group_body(g, gcarry):
            rows = g * 16 + iota
            acc = jnp.zeros((16,), jnp.float32)
            for d in range(D):
                cols = jnp.full((16,), d, jnp.int32)
                sv = plsc.load_gather(s_v, [rows, cols])
                rv = plsc.load_gather(r_v, [rows, cols])
                ov = plsc.load_gather(o_v, [rows, cols])
                acc = acc + sv * rv * ov
            e_v[pl.ds(off + g * 16, 16)] = acc
            return gcarry

        lax.fori_loop(0, NG, group_body, 0)
        return carry

    lax.fori_loop(0, NCHUNK, chunk_body, 0)
    pltpu.sync_copy(e_v, out_hbm.at[pl.ds(base, BPW)])


def _sc_energies(xs, xr, xo, subj, rel, obj):
    mesh = plsc.VectorSubcoreMesh(core_axis_name="c", subcore_axis_name="s",
                                  num_cores=NC, num_subcores=NS)
    kern = pl.kernel(
        _sc_energies_body,
        out_type=jax.ShapeDtypeStruct((B,), jnp.float32),
        mesh=mesh,
        scratch_types=[
            pltpu.VMEM((BPW,), jnp.int32),
            pltpu.VMEM((BPW,), jnp.int32),
            pltpu.VMEM((BPW,), jnp.int32),
            pltpu.VMEM((CH, D), jnp.float32),
            pltpu.VMEM((CH, D), jnp.float32),
            pltpu.VMEM((CH, D), jnp.float32),
            pltpu.VMEM((BPW,), jnp.float32),
            pltpu.SemaphoreType.DMA,
        ],
        compiler_params=pltpu.CompilerParams(needs_layout_passes=False),
    )
    return kern(xs, xr, xo, subj, rel, obj)


def _loss_body(e_ref, y_ref, o_ref):
    x = e_ref[...]
    y = y_ref[...]
    t = (1.0 - y) * x + jnp.log1p(jnp.exp(-jnp.abs(x))) + jnp.maximum(-x, 0.0)
    o_ref[0, 0] = jnp.sum(t) * (1.0 / B)


def _tc_loss(energies, Y):
    out = pl.pallas_call(
        _loss_body,
        out_shape=jax.ShapeDtypeStruct((1, 1), jnp.float32),
        out_specs=pl.BlockSpec(memory_space=pltpu.SMEM),
    )(energies.reshape(B // D, D), Y.reshape(B // D, D))
    return out[0, 0]


@jax.jit
def kernel(X, Y, subject_codes, relation_codes, object_codes):
    xs = X[:, 0]
    xr = X[:, 1]
    xo = X[:, 2]
    energies = _sc_energies(xs, xr, xo, subject_codes, relation_codes,
                            object_codes)
    return _tc_loss(energies, Y)


# speedup vs baseline: 2.0823x; 2.0823x over previous
import functools

import jax
import jax.numpy as jnp
from jax import lax
from jax.experimental import pallas as pl
from jax.experimental.pallas import tpu as pltpu
from jax.experimental.pallas import tpu_sc as plsc

B = 16384
D = 128
NC = 2
NS = 16
NW = NC * NS
BPW = B // NW
CH = 128
NCHUNK = BPW // CH
NG = CH // 16


def _sc_energies_body(xs_hbm, xr_hbm, xo_hbm, subj_hbm, rel_hbm, obj_hbm,
                      out_hbm, xs_v, xr_v, xo_v, s_v, r_v, o_v, e_v, sem):
    wid = lax.axis_index("s") * NC + lax.axis_index("c")
    base = wid * BPW
    pltpu.sync_copy(xs_hbm.at[pl.ds(base, BPW)], xs_v)
    pltpu.sync_copy(xr_hbm.at[pl.ds(base, BPW)], xr_v)
    pltpu.sync_copy(xo_hbm.at[pl.ds(base, BPW)], xo_v)
    iota = lax.iota(jnp.int32, 16)

    def chunk_body(c, carry):
        off = c * CH
        cp_s = pltpu.async_copy(subj_hbm.at[xs_v.at[pl.ds(off, CH)]], s_v, sem)
        cp_r = pltpu.async_copy(rel_hbm.at[xr_v.at[pl.ds(off, CH)]], r_v, sem)
        cp_o = pltpu.async_copy(obj_hbm.at[xo_v.at[pl.ds(off, CH)]], o_v, sem)
        cp_s.wait()
        cp_r.wait()
        cp_o.wait()

        def group_body(g, gcarry):
            rows = g * 16 + iota
            acc = jnp.zeros((16,), jnp.float32)
            for d in range(D):
                cols = (iota + d) & (D - 1)
                sv = plsc.load_gather(s_v, [rows, cols])
                rv = plsc.load_gather(r_v, [rows, cols])
                ov = plsc.load_gather(o_v, [rows, cols])
                acc = acc + sv * rv * ov
            e_v[pl.ds(off + g * 16, 16)] = acc
            return gcarry

        lax.fori_loop(0, NG, group_body, 0)
        return carry

    lax.fori_loop(0, NCHUNK, chunk_body, 0)
    pltpu.sync_copy(e_v, out_hbm.at[pl.ds(base, BPW)])


def _sc_energies(xs, xr, xo, subj, rel, obj):
    mesh = plsc.VectorSubcoreMesh(core_axis_name="c", subcore_axis_name="s",
                                  num_cores=NC, num_subcores=NS)
    kern = pl.kernel(
        _sc_energies_body,
        out_type=jax.ShapeDtypeStruct((B,), jnp.float32),
        mesh=mesh,
        scratch_types=[
            pltpu.VMEM((BPW,), jnp.int32),
            pltpu.VMEM((BPW,), jnp.int32),
            pltpu.VMEM((BPW,), jnp.int32),
            pltpu.VMEM((CH, D), jnp.float32),
            pltpu.VMEM((CH, D), jnp.float32),
            pltpu.VMEM((CH, D), jnp.float32),
            pltpu.VMEM((BPW,), jnp.float32),
            pltpu.SemaphoreType.DMA,
        ],
        compiler_params=pltpu.CompilerParams(needs_layout_passes=False),
    )
    return kern(xs, xr, xo, subj, rel, obj)


def _loss_body(e_ref, y_ref, o_ref):
    x = e_ref[...]
    y = y_ref[...]
    t = (1.0 - y) * x + jnp.log1p(jnp.exp(-jnp.abs(x))) + jnp.maximum(-x, 0.0)
    o_ref[0, 0] = jnp.sum(t) * (1.0 / B)


def _tc_loss(energies, Y):
    out = pl.pallas_call(
        _loss_body,
        out_shape=jax.ShapeDtypeStruct((1, 1), jnp.float32),
        out_specs=pl.BlockSpec(memory_space=pltpu.SMEM),
    )(energies.reshape(B // D, D), Y.reshape(B // D, D))
    return out[0, 0]


@jax.jit
def kernel(X, Y, subject_codes, relation_codes, object_codes):
    xs = X[:, 0]
    xr = X[:, 1]
    xo = X[:, 2]
    energies = _sc_energies(xs, xr, xo, subject_codes, relation_codes,
                            object_codes)
    return _tc_loss(energies, Y)


# speedup vs baseline: 4.0416x; 1.9409x over previous
import functools

import jax
import jax.numpy as jnp
from jax import lax
from jax.experimental import pallas as pl
from jax.experimental.pallas import tpu as pltpu
from jax.experimental.pallas import tpu_sc as plsc

B = 16384
D = 128
NC = 2
NS = 16
NW = NC * NS
BPW = B // NW
CH = 128
NCHUNK = BPW // CH
NG = CH // 16


DBLK = 8


def _sc_energies_body(xs_hbm, xr_hbm, xo_hbm, subj_hbm, rel_hbm, obj_hbm,
                      out_hbm, xs_v, xr_v, xo_v, s_v, r_v, o_v, e_v,
                      sem0, sem1):
    wid = lax.axis_index("s") * NC + lax.axis_index("c")
    base = wid * BPW
    pltpu.sync_copy(xs_hbm.at[pl.ds(base, BPW)], xs_v)
    pltpu.sync_copy(xr_hbm.at[pl.ds(base, BPW)], xr_v)
    pltpu.sync_copy(xo_hbm.at[pl.ds(base, BPW)], xo_v)
    iota = lax.iota(jnp.int32, 16)
    sems = (sem0, sem1)

    def start_chunk(c, slot):
        off = c * CH
        return (
            pltpu.async_copy(subj_hbm.at[xs_v.at[pl.ds(off, CH)]],
                             s_v.at[slot], sems[slot]),
            pltpu.async_copy(rel_hbm.at[xr_v.at[pl.ds(off, CH)]],
                             r_v.at[slot], sems[slot]),
            pltpu.async_copy(obj_hbm.at[xo_v.at[pl.ds(off, CH)]],
                             o_v.at[slot], sems[slot]),
        )

    pending = start_chunk(0, 0)
    for c in range(NCHUNK):
        slot = c % 2
        if c + 1 < NCHUNK:
            nxt = start_chunk(c + 1, 1 - slot)
        for cp in pending:
            cp.wait()
        sb, rb, ob = s_v.at[slot], r_v.at[slot], o_v.at[slot]

        def group_body(g, gcarry, sb=sb, rb=rb, ob=ob, off=c * CH):
            rows = g * 16 + iota

            def dblk_body(k, acc):
                d0 = k * DBLK
                for kk in range(DBLK):
                    cols = (iota + (d0 + kk)) & (D - 1)
                    sv = plsc.load_gather(sb, [rows, cols])
                    rv = plsc.load_gather(rb, [rows, cols])
                    ov = plsc.load_gather(ob, [rows, cols])
                    acc = acc + sv * rv * ov
                return acc

            acc = lax.fori_loop(0, D // DBLK, dblk_body,
                                jnp.zeros((16,), jnp.float32))
            e_v[pl.ds(off + g * 16, 16)] = acc
            return gcarry

        lax.fori_loop(0, NG, group_body, 0)
        if c + 1 < NCHUNK:
            pending = nxt
    pltpu.sync_copy(e_v, out_hbm.at[pl.ds(base, BPW)])


def _sc_energies(xs, xr, xo, subj, rel, obj):
    mesh = plsc.VectorSubcoreMesh(core_axis_name="c", subcore_axis_name="s",
                                  num_cores=NC, num_subcores=NS)
    kern = pl.kernel(
        _sc_energies_body,
        out_type=jax.ShapeDtypeStruct((B,), jnp.float32),
        mesh=mesh,
        scratch_types=[
            pltpu.VMEM((BPW,), jnp.int32),
            pltpu.VMEM((BPW,), jnp.int32),
            pltpu.VMEM((BPW,), jnp.int32),
            pltpu.VMEM((2, CH, D), jnp.float32),
            pltpu.VMEM((2, CH, D), jnp.float32),
            pltpu.VMEM((2, CH, D), jnp.float32),
            pltpu.VMEM((BPW,), jnp.float32),
            pltpu.SemaphoreType.DMA,
            pltpu.SemaphoreType.DMA,
        ],
        compiler_params=pltpu.CompilerParams(needs_layout_passes=False),
    )
    return kern(xs, xr, xo, subj, rel, obj)


def _loss_body(e_ref, y_ref, o_ref):
    x = e_ref[...]
    y = y_ref[...]
    t = (1.0 - y) * x + jnp.log1p(jnp.exp(-jnp.abs(x))) + jnp.maximum(-x, 0.0)
    o_ref[0, 0] = jnp.sum(t) * (1.0 / B)


def _tc_loss(energies, Y):
    out = pl.pallas_call(
        _loss_body,
        out_shape=jax.ShapeDtypeStruct((1, 1), jnp.float32),
        out_specs=pl.BlockSpec(memory_space=pltpu.SMEM),
    )(energies.reshape(B // D, D), Y.reshape(B // D, D))
    return out[0, 0]


@jax.jit
def kernel(X, Y, subject_codes, relation_codes, object_codes):
    xs = X[:, 0]
    xr = X[:, 1]
    xo = X[:, 2]
    energies = _sc_energies(xs, xr, xo, subject_codes, relation_codes,
                            object_codes)
    return _tc_loss(energies, Y)
